# baseline (speedup 1.0000x reference)
import functools
import jax
import jax.numpy as jnp
from jax import lax
from jax.experimental import pallas as pl
from jax.experimental.pallas import tpu as pltpu
from jax.experimental.pallas import tpu_sc as plsc

N = 10000
E = 320000
D = 128
G = 64

NCORES = 2
NSUB = 16
C = 32
NCHT = E // C
CPT_LO = NCHT // (NCORES * NSUB)
CPT_XTRA = NCHT % (NCORES * NSUB)
NP = 10240
NPT = NP // NSUB
ZFULL = NPT // C
NQUAD = (CPT_LO + 1 + 3) // 4

_sc_mesh = plsc.VectorSubcoreMesh(core_axis_name="c", subcore_axis_name="s")


@functools.partial(
    pl.kernel,
    out_type=jax.ShapeDtypeStruct((2, NP, D), jnp.float32),
    mesh=_sc_mesh,
    scratch_types=[
        pltpu.VMEM((C,), jnp.int32),
        pltpu.VMEM((C,), jnp.int32),
        pltpu.VMEM((C,), jnp.int32),
        pltpu.VMEM((C,), jnp.int32),
        pltpu.VMEM((C,), jnp.int32),
        pltpu.VMEM((C,), jnp.int32),
        pltpu.VMEM((C, D), jnp.float32),
        pltpu.VMEM((C, D), jnp.float32),
        pltpu.VMEM((C, D), jnp.float32),
        pltpu.VMEM((C, D), jnp.float32),
        pltpu.VMEM((C, D), jnp.float32),
        pltpu.VMEM((C, D), jnp.float32),
        pltpu.VMEM_SHARED((NP, D), jnp.float32),
    ] + [pltpu.SemaphoreType.DMA] * 12,
)
def _sc_aggregate(x_hbm, src2_hbm, dst2_hbm, ea_hbm, out_hbm,
                  srcv0, srcv1, dstv0, dstv1, dstv2, dstv3,
                  xr0, xr1, ea0, ea1, mg0, mg1, aggr,
                  si0, si1, sj0, sj1, sj2, sj3,
                  sg0, sg1, se0, se1, ss0, ss1):
    cid = lax.axis_index("c")
    sid = lax.axis_index("s")
    srcv = [srcv0, srcv1]
    dstv = [dstv0, dstv1, dstv2, dstv3]
    xr = [xr0, xr1]
    ea = [ea0, ea1]
    mg = [mg0, mg1]
    si = [si0, si1]
    sj = [sj0, sj1, sj2, sj3]
    sg = [sg0, sg1]
    se = [se0, se1]
    ss = [ss0, ss1]

    def zrow(r, carry):
        for k in range(D // 16):
            mg0[r, pl.ds(k * 16, 16)] = jnp.zeros((16,), jnp.float32)
        return carry
    lax.fori_loop(0, C, zrow, 0)
    row0 = sid * NPT
    def zcopy(j, carry):
        pltpu.sync_copy(mg0, aggr.at[pl.ds(row0 + j * C, C)])
        return carry
    lax.fori_loop(0, ZFULL, zcopy, 0)
    plsc.subcore_barrier()

    wid = cid * NSUB + sid
    c0 = wid * CPT_LO + jnp.minimum(wid, CPT_XTRA)
    cpt = jnp.where(wid < CPT_XTRA, CPT_LO + 1, CPT_LO)

    def load_src(c, b):
        return pltpu.async_copy(src2_hbm.at[c0 + c], srcv[b], si[b])

    def load_dst(c, q):
        return pltpu.async_copy(dst2_hbm.at[c0 + c], dstv[q], sj[q])

    def issue_data(c, b):
        pltpu.async_copy(x_hbm.at[srcv[b]], xr[b], sg[b])
        pltpu.async_copy(ea_hbm.at[pl.ds((c0 + c) * C, C)], ea[b], se[b])

    def wait_data(b):
        pltpu.make_async_copy(x_hbm.at[srcv[b]], xr[b], sg[b]).wait()
        pltpu.make_async_copy(ea_hbm.at[pl.ds(0, C)], ea[b], se[b]).wait()

    def compute(b):
        @plsc.parallel_loop(0, C, 2, unroll=2)
        def crow(r):
            for rr in range(2):
                for k in range(D // 16):
                    sl = pl.ds(k * 16, 16)
                    mg[b][r + rr, sl] = jnp.maximum(
                        xr[b][r + rr, sl] + ea[b][r + rr, sl], 0.0)

    def wait_scatter(b):
        pltpu.make_async_copy(mg[b], aggr.at[dstv[0]], ss[b]).wait()

    load_dst(0, 0)
    load_dst(1, 1)
    ld0 = load_src(0, 0)
    load_src(1, 1)
    ld0.wait()
    issue_data(0, 0)

    def quad(kk, carry):
        for p in range(4):
            b, q = p % 2, p
            t = 4 * kk + p
            @pl.when(t < cpt)
            def _():
                @pl.when(t >= 2)
                def _():
                    wait_scatter(b)
                @pl.when(t + 2 < cpt)
                def _():
                    load_dst(t + 2, (p + 2) % 4)
                wait_data(b)
                @pl.when(t + 2 < cpt)
                def _():
                    load_src(t + 2, b)
                compute(b)
                pltpu.make_async_copy(dst2_hbm.at[0], dstv[q], sj[q]).wait()
                pltpu.async_copy(mg[b], aggr.at[dstv[q]], ss[b], add=True)
                @pl.when(t + 1 < cpt)
                def _():
                    pltpu.make_async_copy(src2_hbm.at[0], srcv[1 - b], si[1 - b]).wait()
                    issue_data(t + 1, 1 - b)
        return carry
    lax.fori_loop(0, NQUAD, quad, 0)

    wait_scatter(0)
    wait_scatter(1)
    plsc.subcore_barrier()

    pltpu.sync_copy(aggr.at[pl.ds(row0, NPT)], out_hbm.at[cid, pl.ds(row0, NPT)])


RB = 1000
NRB = N // RB


def _mlp_body(x_ref, a0_ref, a1_ref, wa_ref, ba_ref, wb_ref, bb_ref, o_ref):
    y = x_ref[...] + a0_ref[0] + a1_ref[0]
    h1 = jnp.maximum(
        jnp.dot(y, wa_ref[...], preferred_element_type=jnp.float32) + ba_ref[...], 0.0)
    h2 = jnp.dot(h1, wb_ref[...], preferred_element_type=jnp.float32) + bb_ref[...]
    o_ref[...] = jnp.maximum(h2, 0.0)


_mlp_call = pl.pallas_call(
    _mlp_body,
    grid=(NRB,),
    in_specs=[
        pl.BlockSpec((RB, D), lambda i: (i, 0)),
        pl.BlockSpec((1, RB, D), lambda i: (0, i, 0)),
        pl.BlockSpec((1, RB, D), lambda i: (1, i, 0)),
        pl.BlockSpec((D, D), lambda i: (0, 0)),
        pl.BlockSpec((1, D), lambda i: (0, 0)),
        pl.BlockSpec((D, D), lambda i: (0, 0)),
        pl.BlockSpec((1, D), lambda i: (0, 0)),
    ],
    out_specs=pl.BlockSpec((RB, D), lambda i: (i, 0)),
    out_shape=jax.ShapeDtypeStruct((N, D), jnp.float32),
)


def _mlp_pool_body(x_ref, a0_ref, a1_ref, wa_ref, ba_ref, wb_ref, bb_ref,
                   batch_ref, o_ref, cnt_ref):
    i = pl.program_id(0)
    y = x_ref[...] + a0_ref[0] + a1_ref[0]
    h1 = jnp.maximum(
        jnp.dot(y, wa_ref[...], preferred_element_type=jnp.float32) + ba_ref[...], 0.0)
    h2 = jnp.dot(h1, wb_ref[...], preferred_element_type=jnp.float32) + bb_ref[...]
    h4 = jnp.maximum(h2, 0.0)

    b = batch_ref[0, 0, :].reshape(1, RB)
    gid = lax.broadcasted_iota(jnp.int32, (G, RB), 0)
    oh = (gid == b).astype(jnp.float32)

    @pl.when(i == 0)
    def _():
        o_ref[...] = jnp.zeros((G, D), jnp.float32)
        cnt_ref[...] = jnp.zeros((G, D), jnp.float32)

    o_ref[...] += jnp.dot(oh, h4, preferred_element_type=jnp.float32)
    cnt_ref[...] += jnp.broadcast_to(
        jnp.sum(oh, axis=1, keepdims=True), (G, D))

    @pl.when(i == NRB - 1)
    def _():
        o_ref[...] = o_ref[...] / jnp.maximum(cnt_ref[...], 1.0)


_mlp_pool_call = pl.pallas_call(
    _mlp_pool_body,
    grid=(NRB,),
    in_specs=[
        pl.BlockSpec((RB, D), lambda i: (i, 0)),
        pl.BlockSpec((1, RB, D), lambda i: (0, i, 0)),
        pl.BlockSpec((1, RB, D), lambda i: (1, i, 0)),
        pl.BlockSpec((D, D), lambda i: (0, 0)),
        pl.BlockSpec((1, D), lambda i: (0, 0)),
        pl.BlockSpec((D, D), lambda i: (0, 0)),
        pl.BlockSpec((1, D), lambda i: (0, 0)),
        pl.BlockSpec((1, 1, RB), lambda i: (i, 0, 0)),
    ],
    out_specs=pl.BlockSpec((G, D), lambda i: (0, 0)),
    out_shape=jax.ShapeDtypeStruct((G, D), jnp.float32),
    scratch_shapes=[pltpu.VMEM((G, D), jnp.float32)],
)


def kernel(x, edge_index, edge_attr, batch,
           W1a, b1a, W1b, b1b,
           W2a, b2a, W2b, b2b,
           W3a, b3a, W3b, b3b,
           W4a, b4a, W4b, b4b):
    src2 = edge_index[0].reshape(NCHT, C)
    dst2 = edge_index[1].reshape(NCHT, C)
    batch3 = batch.reshape(NRB, 1, RB)
    layers = [
        (W1a, b1a.reshape(1, D), W1b, b1b.reshape(1, D)),
        (W2a, b2a.reshape(1, D), W2b, b2b.reshape(1, D)),
        (W3a, b3a.reshape(1, D), W3b, b3b.reshape(1, D)),
        (W4a, b4a.reshape(1, D), W4b, b4b.reshape(1, D)),
    ]
    h = x
    for li, (wa, ba, wb, bb) in enumerate(layers):
        aggr = _sc_aggregate(h, src2, dst2, edge_attr)
        if li < 3:
            h = _mlp_call(h, aggr, aggr, wa, ba, wb, bb)
        else:
            out = _mlp_pool_call(h, aggr, aggr, wa, ba, wb, bb, batch3)
    return out


# speedup vs baseline: 1.3136x; 1.3136x over previous
import functools
import jax
import jax.numpy as jnp
from jax import lax
from jax.experimental import pallas as pl
from jax.experimental.pallas import tpu as pltpu
from jax.experimental.pallas import tpu_sc as plsc

N = 10000
E = 320000
D = 128
G = 64

NCORES = 2
NSUB = 16
C = 64
NCHT = E // C
CPT_LO = NCHT // (NCORES * NSUB)
CPT_XTRA = NCHT % (NCORES * NSUB)
NP = 10240
NPT = NP // NSUB
ZFULL = NPT // C
NQUAD = (CPT_LO + 1 + 3) // 4

_sc_mesh = plsc.VectorSubcoreMesh(core_axis_name="c", subcore_axis_name="s")


@functools.partial(
    pl.kernel,
    out_type=jax.ShapeDtypeStruct((2, NP, D), jnp.float32),
    mesh=_sc_mesh,
    scratch_types=[
        pltpu.VMEM((C,), jnp.int32),
        pltpu.VMEM((C,), jnp.int32),
        pltpu.VMEM((C,), jnp.int32),
        pltpu.VMEM((C,), jnp.int32),
        pltpu.VMEM((C,), jnp.int32),
        pltpu.VMEM((C,), jnp.int32),
        pltpu.VMEM((C, D), jnp.float32),
        pltpu.VMEM((C, D), jnp.float32),
        pltpu.VMEM((C, D), jnp.float32),
        pltpu.VMEM((C, D), jnp.float32),
        pltpu.VMEM_SHARED((NP, D), jnp.float32),
    ] + [pltpu.SemaphoreType.DMA] * 12,
)
def _sc_aggregate(x_hbm, src2_hbm, dst2_hbm, ea_hbm, out_hbm,
                  srcv0, srcv1, dstv0, dstv1, dstv2, dstv3,
                  xr0, xr1, ea0, ea1, aggr,
                  si0, si1, sj0, sj1, sj2, sj3,
                  sg0, sg1, se0, se1, ss0, ss1):
    cid = lax.axis_index("c")
    sid = lax.axis_index("s")
    srcv = [srcv0, srcv1]
    dstv = [dstv0, dstv1, dstv2, dstv3]
    xr = [xr0, xr1]
    ea = [ea0, ea1]
    si = [si0, si1]
    sj = [sj0, sj1, sj2, sj3]
    sg = [sg0, sg1]
    se = [se0, se1]
    ss = [ss0, ss1]

    def zrow(r, carry):
        for k in range(D // 16):
            ea0[r, pl.ds(k * 16, 16)] = jnp.zeros((16,), jnp.float32)
        return carry
    lax.fori_loop(0, C, zrow, 0)
    row0 = sid * NPT
    def zcopy(j, carry):
        pltpu.sync_copy(ea0, aggr.at[pl.ds(row0 + j * C, C)])
        return carry
    lax.fori_loop(0, ZFULL, zcopy, 0)
    plsc.subcore_barrier()

    wid = cid * NSUB + sid
    c0 = wid * CPT_LO + jnp.minimum(wid, CPT_XTRA)
    cpt = jnp.where(wid < CPT_XTRA, CPT_LO + 1, CPT_LO)

    def load_src(c, b):
        return pltpu.async_copy(src2_hbm.at[c0 + c], srcv[b], si[b])

    def load_dst(c, q):
        return pltpu.async_copy(dst2_hbm.at[c0 + c], dstv[q], sj[q])

    def issue_data(c, b):
        pltpu.async_copy(x_hbm.at[srcv[b]], xr[b], sg[b])
        pltpu.async_copy(ea_hbm.at[pl.ds((c0 + c) * C, C)], ea[b], se[b])

    def wait_data(b):
        pltpu.make_async_copy(x_hbm.at[srcv[b]], xr[b], sg[b]).wait()
        pltpu.make_async_copy(ea_hbm.at[pl.ds(0, C)], ea[b], se[b]).wait()

    def compute(b):
        @plsc.parallel_loop(0, C, 1, unroll=4)
        def crow(r):
            for k in range(D // 16):
                sl = pl.ds(k * 16, 16)
                xr[b][r, sl] = jnp.maximum(xr[b][r, sl] + ea[b][r, sl], 0.0)

    def wait_scatter(b):
        pltpu.make_async_copy(xr[b], aggr.at[dstv[0]], ss[b]).wait()

    load_dst(0, 0)
    load_dst(1, 1)
    ld0 = load_src(0, 0)
    load_src(1, 1)
    ld0.wait()
    issue_data(0, 0)

    def quad(kk, carry):
        for p in range(4):
            b, q = p % 2, p
            t = 4 * kk + p
            @pl.when(t < cpt)
            def _():
                @pl.when(t + 2 < cpt)
                def _():
                    load_dst(t + 2, (p + 2) % 4)
                wait_data(b)
                @pl.when(t + 2 < cpt)
                def _():
                    load_src(t + 2, b)
                compute(b)
                pltpu.make_async_copy(dst2_hbm.at[0], dstv[q], sj[q]).wait()
                pltpu.async_copy(xr[b], aggr.at[dstv[q]], ss[b], add=True)
                @pl.when(t + 1 < cpt)
                def _():
                    pltpu.make_async_copy(src2_hbm.at[0], srcv[1 - b], si[1 - b]).wait()
                    @pl.when(t >= 1)
                    def _():
                        wait_scatter(1 - b)
                    issue_data(t + 1, 1 - b)
        return carry
    lax.fori_loop(0, NQUAD, quad, 0)

    wait_scatter(0)
    wait_scatter(1)
    plsc.subcore_barrier()

    pltpu.sync_copy(aggr.at[pl.ds(row0, NPT)], out_hbm.at[cid, pl.ds(row0, NPT)])


RB = 1000
NRB = N // RB


def _mlp_body(x_ref, a0_ref, a1_ref, wa_ref, ba_ref, wb_ref, bb_ref, o_ref):
    y = x_ref[...] + a0_ref[0] + a1_ref[0]
    h1 = jnp.maximum(
        jnp.dot(y, wa_ref[...], preferred_element_type=jnp.float32) + ba_ref[...], 0.0)
    h2 = jnp.dot(h1, wb_ref[...], preferred_element_type=jnp.float32) + bb_ref[...]
    o_ref[...] = jnp.maximum(h2, 0.0)


_mlp_call = pl.pallas_call(
    _mlp_body,
    grid=(NRB,),
    in_specs=[
        pl.BlockSpec((RB, D), lambda i: (i, 0)),
        pl.BlockSpec((1, RB, D), lambda i: (0, i, 0)),
        pl.BlockSpec((1, RB, D), lambda i: (1, i, 0)),
        pl.BlockSpec((D, D), lambda i: (0, 0)),
        pl.BlockSpec((1, D), lambda i: (0, 0)),
        pl.BlockSpec((D, D), lambda i: (0, 0)),
        pl.BlockSpec((1, D), lambda i: (0, 0)),
    ],
    out_specs=pl.BlockSpec((RB, D), lambda i: (i, 0)),
    out_shape=jax.ShapeDtypeStruct((N, D), jnp.float32),
)


def _mlp_pool_body(x_ref, a0_ref, a1_ref, wa_ref, ba_ref, wb_ref, bb_ref,
                   batch_ref, o_ref, cnt_ref):
    i = pl.program_id(0)
    y = x_ref[...] + a0_ref[0] + a1_ref[0]
    h1 = jnp.maximum(
        jnp.dot(y, wa_ref[...], preferred_element_type=jnp.float32) + ba_ref[...], 0.0)
    h2 = jnp.dot(h1, wb_ref[...], preferred_element_type=jnp.float32) + bb_ref[...]
    h4 = jnp.maximum(h2, 0.0)

    b = batch_ref[0, 0, :].reshape(1, RB)
    gid = lax.broadcasted_iota(jnp.int32, (G, RB), 0)
    oh = (gid == b).astype(jnp.float32)

    @pl.when(i == 0)
    def _():
        o_ref[...] = jnp.zeros((G, D), jnp.float32)
        cnt_ref[...] = jnp.zeros((G, D), jnp.float32)

    o_ref[...] += jnp.dot(oh, h4, preferred_element_type=jnp.float32)
    cnt_ref[...] += jnp.broadcast_to(
        jnp.sum(oh, axis=1, keepdims=True), (G, D))

    @pl.when(i == NRB - 1)
    def _():
        o_ref[...] = o_ref[...] / jnp.maximum(cnt_ref[...], 1.0)


_mlp_pool_call = pl.pallas_call(
    _mlp_pool_body,
    grid=(NRB,),
    in_specs=[
        pl.BlockSpec((RB, D), lambda i: (i, 0)),
        pl.BlockSpec((1, RB, D), lambda i: (0, i, 0)),
        pl.BlockSpec((1, RB, D), lambda i: (1, i, 0)),
        pl.BlockSpec((D, D), lambda i: (0, 0)),
        pl.BlockSpec((1, D), lambda i: (0, 0)),
        pl.BlockSpec((D, D), lambda i: (0, 0)),
        pl.BlockSpec((1, D), lambda i: (0, 0)),
        pl.BlockSpec((1, 1, RB), lambda i: (i, 0, 0)),
    ],
    out_specs=pl.BlockSpec((G, D), lambda i: (0, 0)),
    out_shape=jax.ShapeDtypeStruct((G, D), jnp.float32),
    scratch_shapes=[pltpu.VMEM((G, D), jnp.float32)],
)


def kernel(x, edge_index, edge_attr, batch,
           W1a, b1a, W1b, b1b,
           W2a, b2a, W2b, b2b,
           W3a, b3a, W3b, b3b,
           W4a, b4a, W4b, b4b):
    src2 = edge_index[0].reshape(NCHT, C)
    dst2 = edge_index[1].reshape(NCHT, C)
    batch3 = batch.reshape(NRB, 1, RB)
    layers = [
        (W1a, b1a.reshape(1, D), W1b, b1b.reshape(1, D)),
        (W2a, b2a.reshape(1, D), W2b, b2b.reshape(1, D)),
        (W3a, b3a.reshape(1, D), W3b, b3b.reshape(1, D)),
        (W4a, b4a.reshape(1, D), W4b, b4b.reshape(1, D)),
    ]
    h = x
    for li, (wa, ba, wb, bb) in enumerate(layers):
        aggr = _sc_aggregate(h, src2, dst2, edge_attr)
        if li < 3:
            h = _mlp_call(h, aggr, aggr, wa, ba, wb, bb)
        else:
            out = _mlp_pool_call(h, aggr, aggr, wa, ba, wb, bb, batch3)
    return out


# speedup vs baseline: 1.3144x; 1.0006x over previous
import functools
import jax
import jax.numpy as jnp
from jax import lax
from jax.experimental import pallas as pl
from jax.experimental.pallas import tpu as pltpu
from jax.experimental.pallas import tpu_sc as plsc

N = 10000
E = 320000
D = 128
G = 64

NCORES = 2
NSUB = 16
C = 64
NCHT = E // C
CPT_LO = NCHT // (NCORES * NSUB)
CPT_XTRA = NCHT % (NCORES * NSUB)
NP = 10240
NPT = NP // NSUB
ZFULL = NPT // C
NQUAD = (CPT_LO + 1 + 3) // 4

_sc_mesh = plsc.VectorSubcoreMesh(core_axis_name="c", subcore_axis_name="s")


@functools.partial(
    pl.kernel,
    out_type=jax.ShapeDtypeStruct((2, NP, D), jnp.float32),
    mesh=_sc_mesh,
    scratch_types=[
        pltpu.VMEM((C,), jnp.int32),
        pltpu.VMEM((C,), jnp.int32),
        pltpu.VMEM((C,), jnp.int32),
        pltpu.VMEM((C,), jnp.int32),
        pltpu.VMEM((C,), jnp.int32),
        pltpu.VMEM((C,), jnp.int32),
        pltpu.VMEM((C, D), jnp.float32),
        pltpu.VMEM((C, D), jnp.float32),
        pltpu.VMEM((C, D), jnp.float32),
        pltpu.VMEM((C, D), jnp.float32),
        pltpu.VMEM_SHARED((NP, D), jnp.float32),
    ] + [pltpu.SemaphoreType.DMA] * 12,
)
def _sc_aggregate(x_hbm, src2_hbm, dst2_hbm, ea_hbm, out_hbm,
                  srcv0, srcv1, dstv0, dstv1, dstv2, dstv3,
                  xr0, xr1, ea0, ea1, aggr,
                  si0, si1, sj0, sj1, sj2, sj3,
                  sg0, sg1, se0, se1, ss0, ss1):
    cid = lax.axis_index("c")
    sid = lax.axis_index("s")
    srcv = [srcv0, srcv1]
    dstv = [dstv0, dstv1, dstv2, dstv3]
    xr = [xr0, xr1]
    ea = [ea0, ea1]
    si = [si0, si1]
    sj = [sj0, sj1, sj2, sj3]
    sg = [sg0, sg1]
    se = [se0, se1]
    ss = [ss0, ss1]

    def zrow(r, carry):
        for k in range(D // 16):
            ea0[r, pl.ds(k * 16, 16)] = jnp.zeros((16,), jnp.float32)
        return carry
    lax.fori_loop(0, C, zrow, 0)
    row0 = sid * NPT
    def zcopy(j, carry):
        pltpu.sync_copy(ea0, aggr.at[pl.ds(row0 + j * C, C)])
        return carry
    lax.fori_loop(0, ZFULL, zcopy, 0)
    plsc.subcore_barrier()

    wid = cid * NSUB + sid
    c0 = wid * CPT_LO + jnp.minimum(wid, CPT_XTRA)
    cpt = jnp.where(wid < CPT_XTRA, CPT_LO + 1, CPT_LO)

    def load_src(c, b):
        return pltpu.async_copy(src2_hbm.at[c0 + c], srcv[b], si[b])

    def load_dst(c, q):
        return pltpu.async_copy(dst2_hbm.at[c0 + c], dstv[q], sj[q])

    def issue_data(c, b):
        pltpu.async_copy(x_hbm.at[srcv[b]], xr[b], sg[b])
        pltpu.async_copy(ea_hbm.at[pl.ds((c0 + c) * C, C)], ea[b], se[b])

    def wait_data(b):
        pltpu.make_async_copy(x_hbm.at[srcv[b]], xr[b], sg[b]).wait()
        pltpu.make_async_copy(ea_hbm.at[pl.ds(0, C)], ea[b], se[b]).wait()

    def compute(b):
        @plsc.parallel_loop(0, C, 1, unroll=8)
        def crow(r):
            for k in range(D // 16):
                sl = pl.ds(k * 16, 16)
                xr[b][r, sl] = jnp.maximum(xr[b][r, sl] + ea[b][r, sl], 0.0)

    def wait_scatter(b):
        pltpu.make_async_copy(xr[b], aggr.at[dstv[0]], ss[b]).wait()

    load_dst(0, 0)
    load_dst(1, 1)
    ld0 = load_src(0, 0)
    load_src(1, 1)
    ld0.wait()
    issue_data(0, 0)

    def quad(kk, carry):
        for p in range(4):
            b, q = p % 2, p
            t = 4 * kk + p
            @pl.when(t < cpt)
            def _():
                @pl.when(t + 2 < cpt)
                def _():
                    load_dst(t + 2, (p + 2) % 4)
                wait_data(b)
                @pl.when(t + 2 < cpt)
                def _():
                    load_src(t + 2, b)
                compute(b)
                pltpu.make_async_copy(dst2_hbm.at[0], dstv[q], sj[q]).wait()
                pltpu.async_copy(xr[b], aggr.at[dstv[q]], ss[b], add=True)
                @pl.when(t + 1 < cpt)
                def _():
                    pltpu.make_async_copy(src2_hbm.at[0], srcv[1 - b], si[1 - b]).wait()
                    @pl.when(t >= 1)
                    def _():
                        wait_scatter(1 - b)
                    issue_data(t + 1, 1 - b)
        return carry
    lax.fori_loop(0, NQUAD, quad, 0)

    wait_scatter(0)
    wait_scatter(1)
    plsc.subcore_barrier()

    pltpu.sync_copy(aggr.at[pl.ds(row0, NPT)], out_hbm.at[cid, pl.ds(row0, NPT)])


RB = 1000
NRB = N // RB


def _mlp_body(x_ref, a0_ref, a1_ref, wa_ref, ba_ref, wb_ref, bb_ref, o_ref):
    y = x_ref[...] + a0_ref[0] + a1_ref[0]
    h1 = jnp.maximum(
        jnp.dot(y, wa_ref[...], preferred_element_type=jnp.float32) + ba_ref[...], 0.0)
    h2 = jnp.dot(h1, wb_ref[...], preferred_element_type=jnp.float32) + bb_ref[...]
    o_ref[...] = jnp.maximum(h2, 0.0)


_mlp_call = pl.pallas_call(
    _mlp_body,
    grid=(NRB,),
    in_specs=[
        pl.BlockSpec((RB, D), lambda i: (i, 0)),
        pl.BlockSpec((1, RB, D), lambda i: (0, i, 0)),
        pl.BlockSpec((1, RB, D), lambda i: (1, i, 0)),
        pl.BlockSpec((D, D), lambda i: (0, 0)),
        pl.BlockSpec((1, D), lambda i: (0, 0)),
        pl.BlockSpec((D, D), lambda i: (0, 0)),
        pl.BlockSpec((1, D), lambda i: (0, 0)),
    ],
    out_specs=pl.BlockSpec((RB, D), lambda i: (i, 0)),
    out_shape=jax.ShapeDtypeStruct((N, D), jnp.float32),
)


def _mlp_pool_body(x_ref, a0_ref, a1_ref, wa_ref, ba_ref, wb_ref, bb_ref,
                   batch_ref, o_ref, cnt_ref):
    i = pl.program_id(0)
    y = x_ref[...] + a0_ref[0] + a1_ref[0]
    h1 = jnp.maximum(
        jnp.dot(y, wa_ref[...], preferred_element_type=jnp.float32) + ba_ref[...], 0.0)
    h2 = jnp.dot(h1, wb_ref[...], preferred_element_type=jnp.float32) + bb_ref[...]
    h4 = jnp.maximum(h2, 0.0)

    b = batch_ref[0, 0, :].reshape(1, RB)
    gid = lax.broadcasted_iota(jnp.int32, (G, RB), 0)
    oh = (gid == b).astype(jnp.float32)

    @pl.when(i == 0)
    def _():
        o_ref[...] = jnp.zeros((G, D), jnp.float32)
        cnt_ref[...] = jnp.zeros((G, D), jnp.float32)

    o_ref[...] += jnp.dot(oh, h4, preferred_element_type=jnp.float32)
    cnt_ref[...] += jnp.broadcast_to(
        jnp.sum(oh, axis=1, keepdims=True), (G, D))

    @pl.when(i == NRB - 1)
    def _():
        o_ref[...] = o_ref[...] / jnp.maximum(cnt_ref[...], 1.0)


_mlp_pool_call = pl.pallas_call(
    _mlp_pool_body,
    grid=(NRB,),
    in_specs=[
        pl.BlockSpec((RB, D), lambda i: (i, 0)),
        pl.BlockSpec((1, RB, D), lambda i: (0, i, 0)),
        pl.BlockSpec((1, RB, D), lambda i: (1, i, 0)),
        pl.BlockSpec((D, D), lambda i: (0, 0)),
        pl.BlockSpec((1, D), lambda i: (0, 0)),
        pl.BlockSpec((D, D), lambda i: (0, 0)),
        pl.BlockSpec((1, D), lambda i: (0, 0)),
        pl.BlockSpec((1, 1, RB), lambda i: (i, 0, 0)),
    ],
    out_specs=pl.BlockSpec((G, D), lambda i: (0, 0)),
    out_shape=jax.ShapeDtypeStruct((G, D), jnp.float32),
    scratch_shapes=[pltpu.VMEM((G, D), jnp.float32)],
)


def kernel(x, edge_index, edge_attr, batch,
           W1a, b1a, W1b, b1b,
           W2a, b2a, W2b, b2b,
           W3a, b3a, W3b, b3b,
           W4a, b4a, W4b, b4b):
    src2 = edge_index[0].reshape(NCHT, C)
    dst2 = edge_index[1].reshape(NCHT, C)
    batch3 = batch.reshape(NRB, 1, RB)
    layers = [
        (W1a, b1a.reshape(1, D), W1b, b1b.reshape(1, D)),
        (W2a, b2a.reshape(1, D), W2b, b2b.reshape(1, D)),
        (W3a, b3a.reshape(1, D), W3b, b3b.reshape(1, D)),
        (W4a, b4a.reshape(1, D), W4b, b4b.reshape(1, D)),
    ]
    h = x
    for li, (wa, ba, wb, bb) in enumerate(layers):
        aggr = _sc_aggregate(h, src2, dst2, edge_attr)
        if li < 3:
            h = _mlp_call(h, aggr, aggr, wa, ba, wb, bb)
        else:
            out = _mlp_pool_call(h, aggr, aggr, wa, ba, wb, bb, batch3)
    return out
